# baseline (speedup 1.0000x reference)
import functools
import jax
import jax.numpy as jnp
from jax import lax
from jax.experimental import pallas as pl
from jax.experimental.pallas import tpu as pltpu
from jax.experimental.pallas import tpu_sc as plsc

OUT_H = 256
OUT_W = 256
WP = OUT_W // 2
N_BOXES = 1000
NPAD = 1024
L = 16
NTILES = 32
BPT = NPAD // NTILES
UNROLL = 4
NPARAM = 7
PROW = 128


def _bf16_bits(x):
    b = lax.bitcast_convert_type(x, jnp.int32)
    r = b + jnp.int32(0x7FFF) + ((b >> 16) & 1)
    return (r >> 16) & jnp.int32(0xFFFF)


def _tc_body(ce_ref, co_ref, bb_ref, mp_ref, pr_ref):
    bb = bb_ref[...]
    x1 = bb[:, 0:1]
    y1 = bb[:, 1:2]
    x2 = bb[:, 2:3]
    y2 = bb[:, 3:4]
    valid = ((x2 - x1) * (y2 - y1)) != 0.0

    gx = lax.broadcasted_iota(jnp.int32, (N_BOXES, OUT_W), 1).astype(
        jnp.float32) * 2.0 + 1.0
    in_x = (gx >= x1) & (gx <= x2) & valid
    in_y = (gx >= y1) & (gx <= y2) & valid

    gp = lax.broadcasted_iota(jnp.int32, (N_BOXES, WP), 1).astype(
        jnp.float32) * 4.0
    in_xe = ((gp + 1.0) >= x1) & ((gp + 1.0) <= x2) & valid
    in_xo = ((gp + 3.0) >= x1) & ((gp + 3.0) <= x2) & valid

    iyf = in_y.astype(jnp.float32)
    count_e = lax.dot_general(
        iyf, in_xe.astype(jnp.float32),
        dimension_numbers=(((0,), (0,)), ((), ())),
        preferred_element_type=jnp.float32)
    count_o = lax.dot_general(
        iyf, in_xo.astype(jnp.float32),
        dimension_numbers=(((0,), (0,)), ((), ())),
        preferred_element_type=jnp.float32)

    pc_e = 1.0 / (1.0 + jnp.exp(-ce_ref[...]))
    pc_o = 1.0 / (1.0 + jnp.exp(-co_ref[...]))
    m_e = jnp.where((count_e > 0.5) & (count_e < 1.5), pc_e, -1.0)
    m_o = jnp.where((count_o > 0.5) & (count_o < 1.5), pc_o, -1.0)
    mp_ref[...] = _bf16_bits(m_e) | (_bf16_bits(m_o) << 16)

    wi = lax.broadcasted_iota(jnp.int32, (N_BOXES, OUT_W), 1)
    big = jnp.int32(OUT_W)
    x0 = jnp.min(jnp.where(in_x, wi, big), axis=1, keepdims=True)
    x1i = jnp.max(jnp.where(in_x, wi, -1), axis=1, keepdims=True)
    y0 = jnp.min(jnp.where(in_y, wi, big), axis=1, keepdims=True)
    y1i = jnp.max(jnp.where(in_y, wi, -1), axis=1, keepdims=True)
    wcnt = x1i - x0 + 1
    hcnt = y1i - y0 + 1
    ok = valid & (wcnt > 0) & (hcnt > 0)
    wx0 = x0 >> 1
    wcw = (x1i >> 1) - wx0 + 1
    area2 = jnp.where(ok, wcw * hcnt, 0)
    invww = jnp.where(wcw > 0, 1.0 / wcw.astype(jnp.float32), 1.0)
    invww_bits = lax.bitcast_convert_type(invww, jnp.int32)

    packed = jnp.concatenate(
        [jnp.broadcast_to(p, (N_BOXES, L))
         for p in (x0, x1i, y0, wx0, wcw, area2, invww_bits)]
        + [jnp.zeros((N_BOXES, PROW - NPARAM * L), jnp.int32)],
        axis=1)
    pr_ref[0:N_BOXES, :] = packed
    pr_ref[N_BOXES:NPAD, :] = jnp.zeros(
        (NPAD - N_BOXES, PROW), jnp.int32)


_tc_call = pl.pallas_call(
    _tc_body,
    out_shape=(
        jax.ShapeDtypeStruct((OUT_H, WP), jnp.int32),
        jax.ShapeDtypeStruct((NPAD, PROW), jnp.int32),
    ),
    compiler_params=pltpu.CompilerParams(
        fuse_transposed_lhs_in_matmul=True),
)


def _sc_body(m_hbm, pr_hbm, score_hbm, keep_hbm, m_v, pr_v, sc_v, kp_v):
    wid = lax.axis_index("s") * 2 + lax.axis_index("c")

    pltpu.sync_copy(m_hbm, m_v)
    pltpu.sync_copy(pr_hbm.at[pl.ds(wid * (BPT * PROW), BPT * PROW)], pr_v)

    lane = lax.iota(jnp.int32, L)
    lanef = lane.astype(jnp.float32)

    for g in range(BPT // L):
        def box_body(i, carry):
            score_vec, keep_vec = carry
            b = (g * L + i) * PROW
            x0 = pr_v[pl.ds(b + 0 * L, L)]
            x1 = pr_v[pl.ds(b + 1 * L, L)]
            y0 = pr_v[pl.ds(b + 2 * L, L)]
            wx0 = pr_v[pl.ds(b + 3 * L, L)]
            wcw = pr_v[pl.ds(b + 4 * L, L)]
            ar = pr_v[pl.ds(b + 5 * L, L)]
            iw = plsc.bitcast(pr_v[pl.ds(b + 6 * L, L)], jnp.float32)
            area_s = ar[0]

            def cond(c):
                return c[0] < area_s

            def body(c):
                base, ki, kf, acc = c
                vals = []
                for u in range(UNROLL):
                    kiu = ki + (u * L)
                    kfu = kf + float(u * L)
                    q = ((kfu + 0.5) * iw).astype(jnp.int32)
                    r = kiu - q * wcw
                    wd = wx0 + r
                    hh = jnp.minimum(y0 + q, OUT_H - 1)
                    word = plsc.load_gather(m_v, [(hh << 7) + wd])
                    lof = lax.bitcast_convert_type(word << 16, jnp.float32)
                    hif = lax.bitcast_convert_type(
                        word & jnp.int32(-65536), jnp.float32)
                    wd2 = wd << 1
                    v = jnp.maximum(jnp.where(wd2 >= x0, lof, -1.0),
                                    jnp.where(wd2 < x1, hif, -1.0))
                    vals.append(jnp.where(kiu < ar, v, -1.0))
                m01 = jnp.maximum(vals[0], vals[1])
                m23 = jnp.maximum(vals[2], vals[3])
                step = jnp.maximum(m01, m23)
                return (base + L * UNROLL, ki + L * UNROLL,
                        kf + float(L * UNROLL), jnp.maximum(acc, step))

            init = (jnp.int32(0), lane, lanef,
                    jnp.full((L,), -1.0, jnp.float32))
            _, _, _, acc = lax.while_loop(cond, body, init)

            mx = lax.sort(acc)[L - 1]
            sel = lane == i
            score_vec = jnp.where(sel, jnp.maximum(mx, 0.0), score_vec)
            keep_vec = jnp.where(sel & (mx > -0.5),
                                 jnp.float32(1.0), keep_vec)
            return score_vec, keep_vec

        score_vec, keep_vec = lax.fori_loop(
            0, L, box_body,
            (jnp.zeros((L,), jnp.float32), jnp.zeros((L,), jnp.float32)))
        sc_v[pl.ds(g * L, L)] = score_vec
        kp_v[pl.ds(g * L, L)] = keep_vec

    pltpu.sync_copy(sc_v, score_hbm.at[pl.ds(wid * BPT, BPT)])
    pltpu.sync_copy(kp_v, keep_hbm.at[pl.ds(wid * BPT, BPT)])


_sc_call = functools.partial(
    pl.kernel,
    out_type=(
        jax.ShapeDtypeStruct((NPAD,), jnp.float32),
        jax.ShapeDtypeStruct((NPAD,), jnp.float32),
    ),
    mesh=plsc.VectorSubcoreMesh(core_axis_name="c", subcore_axis_name="s",
                                num_cores=2, num_subcores=16),
    compiler_params=pltpu.CompilerParams(needs_layout_passes=False),
    scratch_types=[
        pltpu.VMEM((OUT_H * WP,), jnp.int32),
        pltpu.VMEM((BPT * PROW,), jnp.int32),
        pltpu.VMEM((BPT,), jnp.float32),
        pltpu.VMEM((BPT,), jnp.float32),
    ],
)(_sc_body)


@jax.jit
def kernel(input0, raw_bboxes, bboxes):
    conf = input0.reshape(OUT_H, OUT_W)
    mp, params = _tc_call(conf[:, 0::2], conf[:, 1::2], bboxes)
    scores, keeps = _sc_call(mp.reshape(-1), params.reshape(-1))
    return scores[:N_BOXES], keeps[:N_BOXES] > 0.5


# speedup vs baseline: 1.2920x; 1.2920x over previous
import functools
import jax
import jax.numpy as jnp
from jax import lax
from jax.experimental import pallas as pl
from jax.experimental.pallas import tpu as pltpu
from jax.experimental.pallas import tpu_sc as plsc

OUT_H = 256
OUT_W = 256
WP = OUT_W // 2
N_BOXES = 1000
NPAD = 1024
L = 16
NTILES = 32
BPT = NPAD // NTILES
UNROLL = 4
NPARAM = 7
PROW = 128


def _bf16_bits(x):
    b = lax.bitcast_convert_type(x, jnp.int32)
    r = b + jnp.int32(0x7FFF) + ((b >> 16) & 1)
    return (r >> 16) & jnp.int32(0xFFFF)


def _tc_body(conf_ref, bb_ref, mp_ref, pr_ref):
    bb = bb_ref[...]
    x1 = bb[:, 0:1]
    y1 = bb[:, 1:2]
    x2 = bb[:, 2:3]
    y2 = bb[:, 3:4]
    valid = ((x2 - x1) * (y2 - y1)) != 0.0

    gx = lax.broadcasted_iota(jnp.int32, (N_BOXES, OUT_W), 1).astype(
        jnp.float32) * 2.0 + 1.0
    in_x = (gx >= x1) & (gx <= x2) & valid
    in_y = (gx >= y1) & (gx <= y2) & valid

    gp = lax.broadcasted_iota(jnp.int32, (N_BOXES, WP), 1).astype(
        jnp.float32) * 4.0
    in_xe = ((gp + 1.0) >= x1) & ((gp + 1.0) <= x2) & valid
    in_xo = ((gp + 3.0) >= x1) & ((gp + 3.0) <= x2) & valid

    iyf = in_y.astype(jnp.float32)
    count_e = lax.dot_general(
        iyf, in_xe.astype(jnp.float32),
        dimension_numbers=(((0,), (0,)), ((), ())),
        preferred_element_type=jnp.float32)
    count_o = lax.dot_general(
        iyf, in_xo.astype(jnp.float32),
        dimension_numbers=(((0,), (0,)), ((), ())),
        preferred_element_type=jnp.float32)

    pc = 1.0 / (1.0 + jnp.exp(-conf_ref[0]))
    wsel = lax.broadcasted_iota(jnp.int32, (OUT_W, WP), 0)
    wp2 = lax.broadcasted_iota(jnp.int32, (OUT_W, WP), 1) * 2
    s_e = (wsel == wp2).astype(jnp.float32)
    s_o = (wsel == wp2 + 1).astype(jnp.float32)
    pc_e = jnp.dot(pc, s_e, preferred_element_type=jnp.float32)
    pc_o = jnp.dot(pc, s_o, preferred_element_type=jnp.float32)
    m_e = jnp.where((count_e > 0.5) & (count_e < 1.5), pc_e, -1.0)
    m_o = jnp.where((count_o > 0.5) & (count_o < 1.5), pc_o, -1.0)
    mp_ref[...] = _bf16_bits(m_e) | (_bf16_bits(m_o) << 16)

    wi = lax.broadcasted_iota(jnp.int32, (N_BOXES, OUT_W), 1)
    big = jnp.int32(OUT_W)
    x0 = jnp.min(jnp.where(in_x, wi, big), axis=1, keepdims=True)
    x1i = jnp.max(jnp.where(in_x, wi, -1), axis=1, keepdims=True)
    y0 = jnp.min(jnp.where(in_y, wi, big), axis=1, keepdims=True)
    y1i = jnp.max(jnp.where(in_y, wi, -1), axis=1, keepdims=True)
    wcnt = x1i - x0 + 1
    hcnt = y1i - y0 + 1
    ok = valid & (wcnt > 0) & (hcnt > 0)
    wx0 = x0 >> 1
    wcw = (x1i >> 1) - wx0 + 1
    area2 = jnp.where(ok, wcw * hcnt, 0)
    invww = jnp.where(wcw > 0, 1.0 / wcw.astype(jnp.float32), 1.0)
    invww_bits = lax.bitcast_convert_type(invww, jnp.int32)

    packed = jnp.concatenate(
        [jnp.broadcast_to(p, (N_BOXES, L))
         for p in (x0, x1i, y0, wx0, wcw, area2, invww_bits)]
        + [jnp.zeros((N_BOXES, PROW - NPARAM * L), jnp.int32)],
        axis=1)
    pr_ref[0:N_BOXES, :] = packed
    pr_ref[N_BOXES:NPAD, :] = jnp.zeros(
        (NPAD - N_BOXES, PROW), jnp.int32)


_tc_call = pl.pallas_call(
    _tc_body,
    out_shape=(
        jax.ShapeDtypeStruct((OUT_H, WP), jnp.int32),
        jax.ShapeDtypeStruct((NPAD, PROW), jnp.int32),
    ),
    compiler_params=pltpu.CompilerParams(
        fuse_transposed_lhs_in_matmul=True),
)


def _sc_body(m_hbm, pr_hbm, score_hbm, keep_hbm, m_v, pr_v, sc_v, kp_v):
    wid = lax.axis_index("s") * 2 + lax.axis_index("c")

    pltpu.sync_copy(m_hbm, m_v)
    pltpu.sync_copy(pr_hbm.at[pl.ds(wid * (BPT * PROW), BPT * PROW)], pr_v)

    lane = lax.iota(jnp.int32, L)
    lanef = lane.astype(jnp.float32)

    for g in range(BPT // L):
        def box_body(i, carry):
            score_vec, keep_vec = carry
            b = (g * L + i) * PROW
            x0 = pr_v[pl.ds(b + 0 * L, L)]
            x1 = pr_v[pl.ds(b + 1 * L, L)]
            y0 = pr_v[pl.ds(b + 2 * L, L)]
            wx0 = pr_v[pl.ds(b + 3 * L, L)]
            wcw = pr_v[pl.ds(b + 4 * L, L)]
            ar = pr_v[pl.ds(b + 5 * L, L)]
            iw = plsc.bitcast(pr_v[pl.ds(b + 6 * L, L)], jnp.float32)
            area_s = ar[0]

            def cond(c):
                return c[0] < area_s

            def body(c):
                base, ki, kf, acc = c
                vals = []
                for u in range(UNROLL):
                    kiu = ki + (u * L)
                    kfu = kf + float(u * L)
                    q = ((kfu + 0.5) * iw).astype(jnp.int32)
                    r = kiu - q * wcw
                    wd = wx0 + r
                    hh = jnp.minimum(y0 + q, OUT_H - 1)
                    word = plsc.load_gather(m_v, [(hh << 7) + wd])
                    lof = lax.bitcast_convert_type(word << 16, jnp.float32)
                    hif = lax.bitcast_convert_type(
                        word & jnp.int32(-65536), jnp.float32)
                    wd2 = wd << 1
                    v = jnp.maximum(jnp.where(wd2 >= x0, lof, -1.0),
                                    jnp.where(wd2 < x1, hif, -1.0))
                    vals.append(jnp.where(kiu < ar, v, -1.0))
                m01 = jnp.maximum(vals[0], vals[1])
                m23 = jnp.maximum(vals[2], vals[3])
                step = jnp.maximum(m01, m23)
                return (base + L * UNROLL, ki + L * UNROLL,
                        kf + float(L * UNROLL), jnp.maximum(acc, step))

            init = (jnp.int32(0), lane, lanef,
                    jnp.full((L,), -1.0, jnp.float32))
            _, _, _, acc = lax.while_loop(cond, body, init)

            mx = lax.sort(acc)[L - 1]
            sel = lane == i
            score_vec = jnp.where(sel, jnp.maximum(mx, 0.0), score_vec)
            keep_vec = jnp.where(sel & (mx > -0.5),
                                 jnp.float32(1.0), keep_vec)
            return score_vec, keep_vec

        score_vec, keep_vec = lax.fori_loop(
            0, L, box_body,
            (jnp.zeros((L,), jnp.float32), jnp.zeros((L,), jnp.float32)))
        sc_v[pl.ds(g * L, L)] = score_vec
        kp_v[pl.ds(g * L, L)] = keep_vec

    pltpu.sync_copy(sc_v, score_hbm.at[pl.ds(wid * BPT, BPT)])
    pltpu.sync_copy(kp_v, keep_hbm.at[pl.ds(wid * BPT, BPT)])


_sc_call = functools.partial(
    pl.kernel,
    out_type=(
        jax.ShapeDtypeStruct((NPAD,), jnp.float32),
        jax.ShapeDtypeStruct((NPAD,), jnp.float32),
    ),
    mesh=plsc.VectorSubcoreMesh(core_axis_name="c", subcore_axis_name="s",
                                num_cores=2, num_subcores=16),
    compiler_params=pltpu.CompilerParams(needs_layout_passes=False),
    scratch_types=[
        pltpu.VMEM((OUT_H * WP,), jnp.int32),
        pltpu.VMEM((BPT * PROW,), jnp.int32),
        pltpu.VMEM((BPT,), jnp.float32),
        pltpu.VMEM((BPT,), jnp.float32),
    ],
)(_sc_body)


@jax.jit
def kernel(input0, raw_bboxes, bboxes):
    mp, params = _tc_call(input0, bboxes)
    scores, keeps = _sc_call(mp.reshape(-1), params.reshape(-1))
    return scores[:N_BOXES], keeps[:N_BOXES] > 0.5


# speedup vs baseline: 1.4175x; 1.0971x over previous
import functools
import jax
import jax.numpy as jnp
from jax import lax
from jax.experimental import pallas as pl
from jax.experimental.pallas import tpu as pltpu
from jax.experimental.pallas import tpu_sc as plsc

OUT_H = 256
OUT_W = 256
WP = OUT_W // 2
N_BOXES = 1000
NPAD = 1024
L = 16
NTILES = 32
BPT = NPAD // NTILES
UNROLL = 4
NPARAM = 7
PROW = 128


def _bf16_bits(x):
    b = lax.bitcast_convert_type(x, jnp.int32)
    r = b + jnp.int32(0x7FFF) + ((b >> 16) & 1)
    return (r >> 16) & jnp.int32(0xFFFF)


def _tc_body(conf_ref, bb_ref, mp_ref, pr_ref):
    bb = bb_ref[...]
    x1 = bb[:, 0:1]
    y1 = bb[:, 1:2]
    x2 = bb[:, 2:3]
    y2 = bb[:, 3:4]
    valid = ((x2 - x1) * (y2 - y1)) != 0.0

    gx = lax.broadcasted_iota(jnp.int32, (N_BOXES, OUT_W), 1).astype(
        jnp.float32) * 2.0 + 1.0
    in_x = (gx >= x1) & (gx <= x2) & valid
    in_y = (gx >= y1) & (gx <= y2) & valid

    gp = lax.broadcasted_iota(jnp.int32, (N_BOXES, WP), 1).astype(
        jnp.float32) * 4.0
    in_xe = ((gp + 1.0) >= x1) & ((gp + 1.0) <= x2) & valid
    in_xo = ((gp + 3.0) >= x1) & ((gp + 3.0) <= x2) & valid

    iyf = in_y.astype(jnp.float32)
    count_e = lax.dot_general(
        iyf, in_xe.astype(jnp.float32),
        dimension_numbers=(((0,), (0,)), ((), ())),
        preferred_element_type=jnp.float32)
    count_o = lax.dot_general(
        iyf, in_xo.astype(jnp.float32),
        dimension_numbers=(((0,), (0,)), ((), ())),
        preferred_element_type=jnp.float32)

    pc = 1.0 / (1.0 + jnp.exp(-conf_ref[0]))
    wsel = lax.broadcasted_iota(jnp.int32, (OUT_W, WP), 0)
    wp2 = lax.broadcasted_iota(jnp.int32, (OUT_W, WP), 1) * 2
    s_e = (wsel == wp2).astype(jnp.float32)
    s_o = (wsel == wp2 + 1).astype(jnp.float32)
    pc_e = jnp.dot(pc, s_e, preferred_element_type=jnp.float32)
    pc_o = jnp.dot(pc, s_o, preferred_element_type=jnp.float32)
    m_e = jnp.where((count_e > 0.5) & (count_e < 1.5), pc_e, -1.0)
    m_o = jnp.where((count_o > 0.5) & (count_o < 1.5), pc_o, -1.0)
    mp_ref[...] = _bf16_bits(m_e) | (_bf16_bits(m_o) << 16)

    wi = lax.broadcasted_iota(jnp.int32, (N_BOXES, OUT_W), 1)
    big = jnp.int32(OUT_W)
    x0 = jnp.min(jnp.where(in_x, wi, big), axis=1, keepdims=True)
    x1i = jnp.max(jnp.where(in_x, wi, -1), axis=1, keepdims=True)
    y0 = jnp.min(jnp.where(in_y, wi, big), axis=1, keepdims=True)
    y1i = jnp.max(jnp.where(in_y, wi, -1), axis=1, keepdims=True)
    wcnt = x1i - x0 + 1
    hcnt = y1i - y0 + 1
    ok = valid & (wcnt > 0) & (hcnt > 0)
    wx0 = x0 >> 1
    wcw = (x1i >> 1) - wx0 + 1
    area2 = jnp.where(ok, wcw * hcnt, 0)
    invww = jnp.where(wcw > 0, 1.0 / wcw.astype(jnp.float32), 1.0)
    invww_bits = lax.bitcast_convert_type(invww, jnp.int32)

    packed = jnp.concatenate(
        [jnp.broadcast_to(p, (N_BOXES, L))
         for p in (x0, x1i, y0, wx0, wcw, area2, invww_bits)]
        + [jnp.zeros((N_BOXES, PROW - NPARAM * L), jnp.int32)],
        axis=1)
    pr_ref[0:N_BOXES, :] = packed
    pr_ref[N_BOXES:NPAD, :] = jnp.zeros(
        (NPAD - N_BOXES, PROW), jnp.int32)


_tc_call = pl.pallas_call(
    _tc_body,
    out_shape=(
        jax.ShapeDtypeStruct((OUT_H, WP), jnp.int32),
        jax.ShapeDtypeStruct((NPAD, PROW), jnp.int32),
    ),
    compiler_params=pltpu.CompilerParams(
        fuse_transposed_lhs_in_matmul=True),
)


def _sc_body(m_hbm, pr_hbm, score_hbm, keep_hbm, m_sh, m_v, pr_v, sc_v, kp_v):
    sid = lax.axis_index("s")
    wid = sid * 2 + lax.axis_index("c")

    @pl.when(sid == 0)
    def _():
        pltpu.sync_copy(m_hbm, m_sh)
    pltpu.sync_copy(pr_hbm.at[pl.ds(wid * (BPT * PROW), BPT * PROW)], pr_v)
    plsc.subcore_barrier()
    pltpu.sync_copy(m_sh, m_v)

    lane = lax.iota(jnp.int32, L)
    lanef = lane.astype(jnp.float32)

    for g in range(BPT // L):
        def box_body(i, carry):
            score_vec, keep_vec = carry
            b = (g * L + i) * PROW
            x0 = pr_v[pl.ds(b + 0 * L, L)]
            x1 = pr_v[pl.ds(b + 1 * L, L)]
            y0 = pr_v[pl.ds(b + 2 * L, L)]
            wx0 = pr_v[pl.ds(b + 3 * L, L)]
            wcw = pr_v[pl.ds(b + 4 * L, L)]
            ar = pr_v[pl.ds(b + 5 * L, L)]
            iw = plsc.bitcast(pr_v[pl.ds(b + 6 * L, L)], jnp.float32)
            area_s = ar[0]

            def cond(c):
                return c[0] < area_s

            def body(c):
                base, ki, kf, acc = c
                vals = []
                for u in range(UNROLL):
                    kiu = ki + (u * L)
                    kfu = kf + float(u * L)
                    q = ((kfu + 0.5) * iw).astype(jnp.int32)
                    r = kiu - q * wcw
                    wd = wx0 + r
                    hh = jnp.minimum(y0 + q, OUT_H - 1)
                    word = plsc.load_gather(m_v, [(hh << 7) + wd])
                    lof = lax.bitcast_convert_type(word << 16, jnp.float32)
                    hif = lax.bitcast_convert_type(
                        word & jnp.int32(-65536), jnp.float32)
                    wd2 = wd << 1
                    v = jnp.maximum(jnp.where(wd2 >= x0, lof, -1.0),
                                    jnp.where(wd2 < x1, hif, -1.0))
                    vals.append(jnp.where(kiu < ar, v, -1.0))
                m01 = jnp.maximum(vals[0], vals[1])
                m23 = jnp.maximum(vals[2], vals[3])
                step = jnp.maximum(m01, m23)
                return (base + L * UNROLL, ki + L * UNROLL,
                        kf + float(L * UNROLL), jnp.maximum(acc, step))

            init = (jnp.int32(0), lane, lanef,
                    jnp.full((L,), -1.0, jnp.float32))
            _, _, _, acc = lax.while_loop(cond, body, init)

            mx = lax.sort(acc)[L - 1]
            sel = lane == i
            score_vec = jnp.where(sel, jnp.maximum(mx, 0.0), score_vec)
            keep_vec = jnp.where(sel & (mx > -0.5),
                                 jnp.float32(1.0), keep_vec)
            return score_vec, keep_vec

        score_vec, keep_vec = lax.fori_loop(
            0, L, box_body,
            (jnp.zeros((L,), jnp.float32), jnp.zeros((L,), jnp.float32)))
        sc_v[pl.ds(g * L, L)] = score_vec
        kp_v[pl.ds(g * L, L)] = keep_vec

    pltpu.sync_copy(sc_v, score_hbm.at[pl.ds(wid * BPT, BPT)])
    pltpu.sync_copy(kp_v, keep_hbm.at[pl.ds(wid * BPT, BPT)])


_sc_call = functools.partial(
    pl.kernel,
    out_type=(
        jax.ShapeDtypeStruct((NPAD,), jnp.float32),
        jax.ShapeDtypeStruct((NPAD,), jnp.float32),
    ),
    mesh=plsc.VectorSubcoreMesh(core_axis_name="c", subcore_axis_name="s",
                                num_cores=2, num_subcores=16),
    compiler_params=pltpu.CompilerParams(needs_layout_passes=False),
    scratch_types=[
        pltpu.VMEM_SHARED((OUT_H * WP,), jnp.int32),
        pltpu.VMEM((OUT_H * WP,), jnp.int32),
        pltpu.VMEM((BPT * PROW,), jnp.int32),
        pltpu.VMEM((BPT,), jnp.float32),
        pltpu.VMEM((BPT,), jnp.float32),
    ],
)(_sc_body)


@jax.jit
def kernel(input0, raw_bboxes, bboxes):
    mp, params = _tc_call(input0, bboxes)
    scores, keeps = _sc_call(mp.reshape(-1), params.reshape(-1))
    return scores[:N_BOXES], keeps[:N_BOXES] > 0.5
